# baseline (speedup 1.0000x reference)
import jax
import jax.numpy as jnp
from jax.experimental import pallas as pl
from jax.experimental.pallas import tpu as pltpu

B = 16384
D = 128
L = 1024
K = 163
CHUNK = 128
GRID = L // CHUNK


def _body(x_ref, we_ref, be_ref, wd_ref, bd_ref, dec_ref, sparse_ref):
    j = pl.program_id(0)

    x = x_ref[...]
    we = we_ref[...]
    enc = jax.lax.dot_general(
        x, we, (((1,), (1,)), ((), ())),
        preferred_element_type=jnp.float32)
    enc = enc + be_ref[...]

    bits = jax.lax.bitcast_convert_type(enc, jnp.int32)
    m = jnp.where(bits < 0, bits ^ jnp.int32(0x7FFFFFFF), bits)

    ones = jnp.ones((1, B), jnp.bfloat16)

    def count_ge(arr, t):
        ge = (arr >= t).astype(jnp.bfloat16)
        return jax.lax.dot_general(
            ones, ge, (((1,), (0,)), ((), ())),
            preferred_element_type=jnp.float32)

    mh = (m >> 16).astype(jnp.int16)

    kf = jnp.float32(K)
    i32min16 = jnp.int32(-32768)
    h = jnp.where(count_ge(mh, jnp.int16(0)) >= kf, jnp.int32(0), i32min16)
    for b in range(14, -1, -1):
        hc = h + jnp.int32(1 << b)
        h = jnp.where(count_ge(mh, hc.astype(jnp.int16)) >= kf, hc, h)

    h16 = h.astype(jnp.int16)
    in_bucket = mh == h16
    base = jax.lax.dot_general(
        ones, (mh > h16).astype(jnp.bfloat16), (((1,), (0,)), ((), ())),
        preferred_element_type=jnp.float32)
    kf2 = kf - base
    ml = ((m & jnp.int32(0xFFFF)) ^ jnp.int32(0x8000)).astype(jnp.int16)
    mlm = jnp.where(in_bucket, ml, jnp.int16(-32768))

    low = jnp.where(count_ge(mlm, jnp.int16(0)) >= kf2, jnp.int32(0), i32min16)
    for b in range(14, -1, -1):
        lc = low + jnp.int32(1 << b)
        low = jnp.where(count_ge(mlm, lc.astype(jnp.int16)) >= kf2, lc, low)

    t = (h << 16) | ((low ^ jnp.int32(0x8000)) & jnp.int32(0xFFFF))

    sp = jnp.where(m >= t, enc, 0.0)
    sparse_ref[...] = sp

    part = jax.lax.dot_general(
        sp, wd_ref[...], (((1,), (1,)), ((), ())),
        preferred_element_type=jnp.float32)

    @pl.when(j == 0)
    def _():
        dec_ref[...] = part + bd_ref[...]

    @pl.when(j > 0)
    def _():
        dec_ref[...] = dec_ref[...] + part


@jax.jit
def kernel(x, W_enc, b_enc, W_dec, b_dec):
    decoded, sparse = pl.pallas_call(
        _body,
        grid=(GRID,),
        in_specs=[
            pl.BlockSpec((B, D), lambda j: (0, 0)),
            pl.BlockSpec((CHUNK, D), lambda j: (j, 0)),
            pl.BlockSpec((1, CHUNK), lambda j: (0, j)),
            pl.BlockSpec((D, CHUNK), lambda j: (0, j)),
            pl.BlockSpec((1, D), lambda j: (0, 0)),
        ],
        out_specs=[
            pl.BlockSpec((B, D), lambda j: (0, 0)),
            pl.BlockSpec((B, CHUNK), lambda j: (0, j)),
        ],
        out_shape=[
            jax.ShapeDtypeStruct((B, D), jnp.float32),
            jax.ShapeDtypeStruct((B, L), jnp.float32),
        ],
        compiler_params=pltpu.CompilerParams(
            vmem_limit_bytes=64 * 1024 * 1024),
    )(x, W_enc, b_enc.reshape(1, L), W_dec, b_dec.reshape(1, D))
    return (decoded, sparse)


# speedup vs baseline: 2.1823x; 2.1823x over previous
import jax
import jax.numpy as jnp
from jax.experimental import pallas as pl
from jax.experimental.pallas import tpu as pltpu

B = 16384
D = 128
L = 1024
K = 163
CHUNK = 128
GRID = L // CHUNK


def _body(x_ref, we_ref, be_ref, wd_ref, bd_ref, dec_ref, sparse_ref):
    j = pl.program_id(0)

    x = x_ref[...]
    we = we_ref[...]
    enc = jax.lax.dot_general(
        x, we, (((1,), (1,)), ((), ())),
        preferred_element_type=jnp.float32)
    enc = enc + be_ref[...]

    bits = jax.lax.bitcast_convert_type(enc, jnp.int32)
    m = jnp.where(bits < 0, bits ^ jnp.int32(0x7FFFFFFF), bits)

    ones = jnp.ones((1, B), jnp.bfloat16)

    def count_ge(arr, t):
        ge = jax.lax.bitcast_convert_type(
            jnp.where(arr >= t, jnp.int16(0x3F80), jnp.int16(0)),
            jnp.bfloat16)
        return jax.lax.dot_general(
            ones, ge, (((1,), (0,)), ((), ())),
            preferred_element_type=jnp.float32)

    mh = (m >> 16).astype(jnp.int16)

    def search16(arr, kneed):
        c3 = count_ge(arr, jnp.int16(16384))
        c2 = count_ge(arr, jnp.int16(0))
        c1 = count_ge(arr, jnp.int16(-16384))
        t = jnp.where(
            c3 >= kneed, jnp.int32(16384),
            jnp.where(c2 >= kneed, jnp.int32(0),
                      jnp.where(c1 >= kneed, jnp.int32(-16384),
                                jnp.int32(-32768))))
        for b in range(12, -1, -2):
            d1 = jnp.int32(1 << b)
            d2 = jnp.int32(2 << b)
            c3 = count_ge(arr, (t + d2 + d1).astype(jnp.int16))
            c2 = count_ge(arr, (t + d2).astype(jnp.int16))
            c1 = count_ge(arr, (t + d1).astype(jnp.int16))
            t = t + jnp.where(
                c3 >= kneed, d2 + d1,
                jnp.where(c2 >= kneed, d2,
                          jnp.where(c1 >= kneed, d1, jnp.int32(0))))
        return t

    kf = jnp.float32(K)
    h = search16(mh, kf)
    h16 = h.astype(jnp.int16)
    in_bucket = mh == h16
    gt = jax.lax.bitcast_convert_type(
        jnp.where(mh > h16, jnp.int16(0x3F80), jnp.int16(0)), jnp.bfloat16)
    base = jax.lax.dot_general(
        ones, gt, (((1,), (0,)), ((), ())),
        preferred_element_type=jnp.float32)
    kf2 = kf - base
    ml = ((m & jnp.int32(0xFFFF)) ^ jnp.int32(0x8000)).astype(jnp.int16)
    mlm = jnp.where(in_bucket, ml, jnp.int16(-32768))

    low = search16(mlm, kf2)
    t = (h << 16) | ((low ^ jnp.int32(0x8000)) & jnp.int32(0xFFFF))

    sp = jnp.where(m >= t, enc, 0.0)
    sparse_ref[...] = sp

    part = jax.lax.dot_general(
        sp, wd_ref[...], (((1,), (1,)), ((), ())),
        preferred_element_type=jnp.float32)

    @pl.when(j == 0)
    def _():
        dec_ref[...] = part + bd_ref[...]

    @pl.when(j > 0)
    def _():
        dec_ref[...] = dec_ref[...] + part


@jax.jit
def kernel(x, W_enc, b_enc, W_dec, b_dec):
    decoded, sparse = pl.pallas_call(
        _body,
        grid=(GRID,),
        in_specs=[
            pl.BlockSpec((B, D), lambda j: (0, 0)),
            pl.BlockSpec((CHUNK, D), lambda j: (j, 0)),
            pl.BlockSpec((1, CHUNK), lambda j: (0, j)),
            pl.BlockSpec((D, CHUNK), lambda j: (0, j)),
            pl.BlockSpec((1, D), lambda j: (0, 0)),
        ],
        out_specs=[
            pl.BlockSpec((B, D), lambda j: (0, 0)),
            pl.BlockSpec((B, CHUNK), lambda j: (0, j)),
        ],
        out_shape=[
            jax.ShapeDtypeStruct((B, D), jnp.float32),
            jax.ShapeDtypeStruct((B, L), jnp.float32),
        ],
        compiler_params=pltpu.CompilerParams(
            vmem_limit_bytes=64 * 1024 * 1024),
    )(x, W_enc, b_enc.reshape(1, L), W_dec, b_dec.reshape(1, D))
    return (decoded, sparse)


# speedup vs baseline: 2.8207x; 1.2926x over previous
import jax
import jax.numpy as jnp
from jax.experimental import pallas as pl
from jax.experimental.pallas import tpu as pltpu

B = 16384
D = 128
L = 1024
K = 163
CHUNK = 128
GRID = L // CHUNK


def _body(x_ref, we_ref, be_ref, wd_ref, bd_ref, dec_ref, sparse_ref):
    j = pl.program_id(0)

    x = x_ref[...]
    we = we_ref[...]
    enc = jax.lax.dot_general(
        x, we, (((1,), (1,)), ((), ())),
        preferred_element_type=jnp.float32)
    enc = enc + be_ref[...]

    bits = jax.lax.bitcast_convert_type(enc, jnp.int32)
    m = jnp.where(bits < 0, bits ^ jnp.int32(0x7FFFFFFF), bits)

    ones = jnp.ones((1, B), jnp.bfloat16)

    def count_ge(t):
        ge = (m >= t).astype(jnp.bfloat16)
        return jax.lax.dot_general(
            ones, ge, (((1,), (0,)), ((), ())),
            preferred_element_type=jnp.float32)

    kf = jnp.float32(K)
    t = jnp.where(count_ge(jnp.int32(0)) >= kf,
                  jnp.int32(0), jnp.iinfo(jnp.int32).min)
    for b in range(30, -1, -1):
        t_hi = t + jnp.int32(1 << b)
        t = jnp.where(count_ge(t_hi) >= kf, t_hi, t)

    sp = jnp.where(m >= t, enc, 0.0)
    sparse_ref[...] = sp

    part = jax.lax.dot_general(
        sp, wd_ref[...], (((1,), (1,)), ((), ())),
        preferred_element_type=jnp.float32)

    @pl.when(j == 0)
    def _():
        dec_ref[...] = part + bd_ref[...]

    @pl.when(j > 0)
    def _():
        dec_ref[...] = dec_ref[...] + part


@jax.jit
def kernel(x, W_enc, b_enc, W_dec, b_dec):
    decoded, sparse = pl.pallas_call(
        _body,
        grid=(GRID,),
        in_specs=[
            pl.BlockSpec((B, D), lambda j: (0, 0)),
            pl.BlockSpec((CHUNK, D), lambda j: (j, 0)),
            pl.BlockSpec((1, CHUNK), lambda j: (0, j)),
            pl.BlockSpec((D, CHUNK), lambda j: (0, j)),
            pl.BlockSpec((1, D), lambda j: (0, 0)),
        ],
        out_specs=[
            pl.BlockSpec((B, D), lambda j: (0, 0)),
            pl.BlockSpec((B, CHUNK), lambda j: (0, j)),
        ],
        out_shape=[
            jax.ShapeDtypeStruct((B, D), jnp.float32),
            jax.ShapeDtypeStruct((B, L), jnp.float32),
        ],
        compiler_params=pltpu.CompilerParams(
            vmem_limit_bytes=64 * 1024 * 1024),
    )(x, W_enc, b_enc.reshape(1, L), W_dec, b_dec.reshape(1, D))
    return (decoded, sparse)
